# baseline (speedup 1.0000x reference)
import functools

import jax
import jax.numpy as jnp
import numpy as np
from jax import lax
from jax.experimental import pallas as pl
from jax.experimental.pallas import tpu as pltpu
from jax.experimental.pallas import tpu_sc as plsc

N_FIELDS = 26
FIELD_DIM = 100000
EMBED = 32
N_FLOAT = 13
BATCH = 16384

NUM_CORES = 2
NUM_SUBCORES = 16
NW = NUM_CORES * NUM_SUBCORES
B_W = BATCH // NW
ROWS_W = B_W * N_FIELDS
PAT = 208

TROWS = N_FIELDS * FIELD_DIM
CH = 320
NCH = TROWS // CH

CB = 16
NB = B_W // CB
CROWS = CB * N_FIELDS


def _sc_compact(table):
    mesh = plsc.VectorSubcoreMesh(
        core_axis_name="c", subcore_axis_name="s",
        num_cores=NUM_CORES, num_subcores=NUM_SUBCORES)

    @functools.partial(
        pl.kernel,
        mesh=mesh,
        out_type=jax.ShapeDtypeStruct((TROWS // 4, 128), jnp.float32),
        scratch_types=[
            pltpu.VMEM((CH, EMBED), jnp.float32),
            pltpu.VMEM((CH, EMBED), jnp.float32),
            pltpu.VMEM((CH // 4, 128), jnp.float32),
            pltpu.SemaphoreType.DMA,
            pltpu.SemaphoreType.DMA,
        ],
        compiler_params=pltpu.CompilerParams(needs_layout_passes=False),
    )
    def k(tab_hbm, out_hbm, va, vb, vp, sa, sb):
        wid = lax.axis_index("s") * NUM_CORES + lax.axis_index("c")

        def fire(c, buf, sem):
            @pl.when(c < NCH)
            def _():
                pltpu.make_async_copy(
                    tab_hbm.at[pl.ds(c * CH, CH)], buf, sem).start()

        def drain(c, buf, sem):
            @pl.when(c < NCH)
            def _():
                pltpu.make_async_copy(
                    tab_hbm.at[pl.ds(c * CH, CH)], buf, sem).wait()

        def process(c, buf):
            @pl.when(c < NCH)
            def _():
                for r in range(CH):
                    p = r // 4
                    l = (r % 4) * EMBED
                    vp[p, pl.ds(l, 16)] = buf[r, pl.ds(0, 16)]
                    vp[p, pl.ds(l + 16, 16)] = buf[r, pl.ds(16, 16)]
                pltpu.sync_copy(
                    vp, out_hbm.at[pl.ds(c * (CH // 4), CH // 4)])

        fire(wid, va, sa)
        def outer(i, carry):
            cA = wid + i * 64
            cB = cA + 32
            fire(cB, vb, sb)
            drain(cA, va, sa)
            process(cA, va)
            fire(cA + 64, va, sa)
            drain(cB, vb, sb)
            process(cB, vb)
            return carry
        lax.fori_loop(0, (NCH + 63) // 64, outer, 0)

    return k(table)


def _sc_gather(tok_flat, pat, packed2, ff_flat, w_flat, b_flat):
    mesh = plsc.VectorSubcoreMesh(
        core_axis_name="c", subcore_axis_name="s",
        num_cores=NUM_CORES, num_subcores=NUM_SUBCORES)

    @functools.partial(
        pl.kernel,
        mesh=mesh,
        out_type=jax.ShapeDtypeStruct((BATCH, N_FIELDS + 1, EMBED),
                                      jnp.float32),
        scratch_types=[
            pltpu.VMEM((ROWS_W,), jnp.int32),
            pltpu.VMEM((PAT,), jnp.int32),
            pltpu.VMEM((CROWS, EMBED), jnp.float32),
            pltpu.VMEM((CROWS, EMBED), jnp.float32),
            pltpu.VMEM((CB, N_FIELDS + 1, EMBED), jnp.float32),
            pltpu.VMEM((B_W * N_FLOAT,), jnp.float32),
            pltpu.VMEM((N_FLOAT * EMBED,), jnp.float32),
            pltpu.VMEM((EMBED,), jnp.float32),
            pltpu.SemaphoreType.DMA,
            pltpu.SemaphoreType.DMA,
        ],
        compiler_params=pltpu.CompilerParams(
            use_tc_tiling_on_sc=False, needs_layout_passes=False),
    )
    def k(tok_hbm, pat_hbm, table_hbm, ff_hbm, w_hbm, b_hbm, out_hbm,
          idx_v, pat_v, g0, g1, stg, ffb, wb, bb, s0, s1):
        wid = lax.axis_index("s") * NUM_CORES + lax.axis_index("c")
        base = wid * ROWS_W
        b0 = wid * B_W
        pltpu.sync_copy(tok_hbm.at[pl.ds(base, ROWS_W)], idx_v)
        pltpu.sync_copy(pat_hbm, pat_v)
        pltpu.sync_copy(ff_hbm.at[pl.ds(wid * B_W * N_FLOAT, B_W * N_FLOAT)],
                        ffb)
        pltpu.sync_copy(w_hbm, wb)
        pltpu.sync_copy(b_hbm, bb)

        def add_body(g, carry):
            for v in range(PAT // 16):
                sl = pl.ds(g * PAT + v * 16, 16)
                idx_v[sl] = idx_v[sl] + pat_v[pl.ds(v * 16, 16)]
            return carry
        lax.fori_loop(0, ROWS_W // PAT, add_body, 0)

        wlo = [wb[pl.ds(kf * EMBED, 16)] for kf in range(N_FLOAT)]
        whi = [wb[pl.ds(kf * EMBED + 16, 16)] for kf in range(N_FLOAT)]
        blo = bb[pl.ds(0, 16)]
        bhi = bb[pl.ds(16, 16)]
        zeros16 = jnp.zeros((16,), jnp.int32)

        bufs = (g0, g1)
        sems = (s0, s1)

        DLENS = (128, 128, 128, 32)
        DOFFS = (0, 128, 256, 384)

        def dmas(c, p):
            r0 = c * CROWS
            return [pltpu.make_async_copy(
                table_hbm.at[idx_v.at[pl.ds(r0 + o, n)]],
                bufs[p].at[pl.ds(o, n)],
                sems[p]) for o, n in zip(DOFFS, DLENS)]

        def fire(c, p):
            for h in dmas(c, p):
                h.start()

        def drain(c, p):
            for h in dmas(c, p):
                h.wait()

        def process(c, p):
            buf = bufs[p]
            cb0 = c * CB * N_FLOAT
            for i in range(CB):
                for r in range(N_FIELDS):
                    stg[i, r, pl.ds(0, 16)] = buf[i * N_FIELDS + r,
                                                  pl.ds(0, 16)]
                    stg[i, r, pl.ds(16, 16)] = buf[i * N_FIELDS + r,
                                                   pl.ds(16, 16)]
                acc0 = blo
                acc1 = bhi
                for kf in range(N_FLOAT):
                    sidx = zeros16 + (cb0 + i * N_FLOAT + kf)
                    v = plsc.load_gather(ffb, [sidx])
                    acc0 = acc0 + v * wlo[kf]
                    acc1 = acc1 + v * whi[kf]
                stg[i, N_FIELDS, pl.ds(0, 16)] = acc0
                stg[i, N_FIELDS, pl.ds(16, 16)] = acc1
            pltpu.sync_copy(
                stg, out_hbm.at[pl.ds(b0 + c * CB, CB), :, :])

        fire(0, 0)
        def chunk_body(i, carry):
            cA = 2 * i
            cB = 2 * i + 1
            fire(cB, 1)
            drain(cA, 0)
            process(cA, 0)
            @pl.when(cA + 2 < NB)
            def _():
                fire(cA + 2, 0)
            drain(cB, 1)
            process(cB, 1)
            return carry
        lax.fori_loop(0, NB // 2, chunk_body, 0)

    return k(tok_flat, pat, packed2, ff_flat, w_flat, b_flat)


def kernel(token_fields, float_fields, table, W_float, b_float):
    tok_flat = token_fields.astype(jnp.int32).reshape(-1)
    pat = jnp.asarray(
        np.tile(np.arange(N_FIELDS, dtype=np.int32) * FIELD_DIM,
                PAT // N_FIELDS))
    packed = _sc_compact(table)
    packed2 = packed.reshape(TROWS, EMBED)
    ff_flat = float_fields.reshape(-1)
    w_flat = W_float.reshape(-1)
    return _sc_gather(tok_flat, pat, packed2, ff_flat, w_flat, b_float)
